# baseline (speedup 1.0000x reference)
import functools

import jax
import jax.numpy as jnp
from jax import lax
from jax.experimental import pallas as pl
from jax.experimental.pallas import tpu as pltpu
from jax.experimental.pallas import tpu_sc as plsc


def _sqrt_tables(alpha_bar):
    t = alpha_bar.shape[0]
    pad = 1024 - t
    ab = jnp.concatenate([alpha_bar.astype(jnp.float32),
                          jnp.zeros((pad,), jnp.float32)]).reshape(8, 128)

    def body(a_ref, sa_ref, sb_ref):
        a = a_ref[...]
        sa_ref[...] = jnp.sqrt(a)
        sb_ref[...] = jnp.sqrt(jnp.maximum(1.0 - a, 0.0))

    sa, sb = pl.pallas_call(
        body,
        out_shape=(jax.ShapeDtypeStruct((8, 128), jnp.float32),
                   jax.ShapeDtypeStruct((8, 128), jnp.float32)),
    )(ab)
    return sa.reshape(1024), sb.reshape(1024)


def _vreg_take(v, idx):
    dnums = lax.GatherDimensionNumbers(
        offset_dims=(), collapsed_slice_dims=(0,), start_index_map=(0,))
    return lax.gather(v, idx[:, None], dnums, slice_sizes=(1,),
                      mode=lax.GatherScatterMode.PROMISE_IN_BOUNDS)


def _sc_scale_add(z0, t_n, noise, sa, sb):
    n, k = z0.shape
    info = plsc.get_sparse_core_info()
    nc, ns, lanes = info.num_cores, info.num_subcores, info.num_lanes
    nw = nc * ns
    rpw = n // nw
    rc = 128 if rpw % 128 == 0 else rpw
    nchunks = rpw // rc
    nvec = k // lanes
    ngroups = rc // lanes

    mesh = plsc.VectorSubcoreMesh(core_axis_name="c", subcore_axis_name="s")

    @functools.partial(
        pl.kernel,
        mesh=mesh,
        compiler_params=pltpu.CompilerParams(needs_layout_passes=False),
        out_type=jax.ShapeDtypeStruct((n, k), jnp.float32),
        scratch_types=[
            pltpu.VMEM((rc, k), jnp.float32),
            pltpu.VMEM((rc, k), jnp.float32),
            pltpu.VMEM((rc, k), jnp.float32),
            pltpu.VMEM((rc, k), jnp.float32),
            pltpu.VMEM((rc, k), jnp.float32),
            pltpu.VMEM((rc, k), jnp.float32),
            pltpu.VMEM((rpw,), jnp.int32),
            pltpu.VMEM((1024,), jnp.float32),
            pltpu.VMEM((1024,), jnp.float32),
            pltpu.SemaphoreType.DMA,
            pltpu.SemaphoreType.DMA,
            pltpu.SemaphoreType.DMA,
            pltpu.SemaphoreType.DMA,
            pltpu.SemaphoreType.DMA,
            pltpu.SemaphoreType.DMA,
        ],
    )
    def run(z0_h, t_h, nz_h, sa_h, sb_h, out_h,
            z0v0, z0v1, nzv0, nzv1, ov0, ov1, t_v, sa_v, sb_v,
            sz0, sz1, sn0, sn1, so0, so1):
        z0v = (z0v0, z0v1)
        nzv = (nzv0, nzv1)
        ov = (ov0, ov1)
        sz = (sz0, sz1)
        sn = (sn0, sn1)
        so = (so0, so1)
        wid = lax.axis_index("s") * nc + lax.axis_index("c")
        base = wid * rpw
        pltpu.sync_copy(t_h.at[pl.ds(base, rpw)], t_v)
        pltpu.sync_copy(sa_h, sa_v)
        pltpu.sync_copy(sb_h, sb_v)

        def start_in(c):
            b = c % 2
            rb = base + c * rc
            hz = pltpu.async_copy(z0_h.at[pl.ds(rb, rc)], z0v[b], sz[b])
            hn = pltpu.async_copy(nz_h.at[pl.ds(rb, rc)], nzv[b], sn[b])
            return (hz, hn)

        def start_out(c):
            b = c % 2
            rb = base + c * rc
            return pltpu.async_copy(ov[b], out_h.at[pl.ds(rb, rc)], so[b])

        def compute(c):
            b = c % 2
            z0b, nzb, ob = z0v[b], nzv[b], ov[b]

            def group(g, carry):
                row0 = g * lanes
                tb = plsc.load_gather(
                    t_v, [c * rc + row0 + lax.iota(jnp.int32, lanes)])
                a16 = plsc.load_gather(sa_v, [tb])
                b16 = plsc.load_gather(sb_v, [tb])
                for r in range(lanes):
                    idxr = jnp.full((lanes,), r, jnp.int32)
                    ab = _vreg_take(a16, idxr)
                    bb = _vreg_take(b16, idxr)
                    row = row0 + r
                    for j in range(nvec):
                        s = pl.ds(j * lanes, lanes)
                        ob[row, s] = ab * z0b[row, s] + bb * nzb[row, s]
                return carry

            lax.fori_loop(0, ngroups, group, 0)

        in_h = {}
        out_h_d = {}
        in_h[0] = start_in(0)
        if nchunks > 1:
            in_h[1] = start_in(1)
        for c in range(nchunks):
            if c >= 2:
                out_h_d[c - 2].wait()
            in_h[c][0].wait()
            in_h[c][1].wait()
            compute(c)
            out_h_d[c] = start_out(c)
            if c + 2 < nchunks:
                in_h[c + 2] = start_in(c + 2)
        for c in range(max(0, nchunks - 2), nchunks):
            out_h_d[c].wait()

    return run(z0, t_n, noise, sa, sb)


def kernel(z0_nk, t_n, noise, alpha_bar):
    sa, sb = _sqrt_tables(alpha_bar)
    z_t = _sc_scale_add(z0_nk, t_n.astype(jnp.int32), noise, sa, sb)
    return (z_t, noise)


# speedup vs baseline: 1.1930x; 1.1930x over previous
import functools

import jax
import jax.numpy as jnp
from jax import lax
from jax.experimental import pallas as pl
from jax.experimental.pallas import tpu as pltpu
from jax.experimental.pallas import tpu_sc as plsc


def _sqrt_tables(alpha_bar):
    t = alpha_bar.shape[0]
    pad = 1024 - t
    ab = jnp.concatenate([alpha_bar.astype(jnp.float32),
                          jnp.zeros((pad,), jnp.float32)]).reshape(8, 128)

    def body(a_ref, sa_ref, sb_ref):
        a = a_ref[...]
        sa_ref[...] = jnp.sqrt(a)
        sb_ref[...] = jnp.sqrt(jnp.maximum(1.0 - a, 0.0))

    sa, sb = pl.pallas_call(
        body,
        out_shape=(jax.ShapeDtypeStruct((8, 128), jnp.float32),
                   jax.ShapeDtypeStruct((8, 128), jnp.float32)),
    )(ab)
    return sa.reshape(1024), sb.reshape(1024)


def _vreg_take(v, idx):
    dnums = lax.GatherDimensionNumbers(
        offset_dims=(), collapsed_slice_dims=(0,), start_index_map=(0,))
    return lax.gather(v, idx[:, None], dnums, slice_sizes=(1,),
                      mode=lax.GatherScatterMode.PROMISE_IN_BOUNDS)


def _sc_scale_add(z0, t_n, noise, sa, sb):
    n, k = z0.shape
    info = plsc.get_sparse_core_info()
    nc, ns, lanes = info.num_cores, info.num_subcores, info.num_lanes
    nw = nc * ns
    rpw = n // nw
    rc = 128 if rpw % 128 == 0 else rpw
    nchunks = rpw // rc
    nvec = k // lanes
    ngroups = rc // lanes

    mesh = plsc.VectorSubcoreMesh(core_axis_name="c", subcore_axis_name="s")

    @functools.partial(
        pl.kernel,
        mesh=mesh,
        compiler_params=pltpu.CompilerParams(needs_layout_passes=False),
        out_type=jax.ShapeDtypeStruct((n, k), jnp.float32),
        scratch_types=[
            pltpu.VMEM((rc, k), jnp.float32),
            pltpu.VMEM((rc, k), jnp.float32),
            pltpu.VMEM((rc, k), jnp.float32),
            pltpu.VMEM((rc, k), jnp.float32),
            pltpu.VMEM((rc, k), jnp.float32),
            pltpu.VMEM((rc, k), jnp.float32),
            pltpu.VMEM((rpw,), jnp.int32),
            pltpu.VMEM((1024,), jnp.float32),
            pltpu.VMEM((1024,), jnp.float32),
            pltpu.SemaphoreType.DMA,
            pltpu.SemaphoreType.DMA,
            pltpu.SemaphoreType.DMA,
            pltpu.SemaphoreType.DMA,
            pltpu.SemaphoreType.DMA,
            pltpu.SemaphoreType.DMA,
        ],
    )
    def run(z0_h, t_h, nz_h, sa_h, sb_h, out_h,
            z0v0, z0v1, nzv0, nzv1, ov0, ov1, t_v, sa_v, sb_v,
            sz0, sz1, sn0, sn1, so0, so1):
        z0v = (z0v0, z0v1)
        nzv = (nzv0, nzv1)
        ov = (ov0, ov1)
        sz = (sz0, sz1)
        sn = (sn0, sn1)
        so = (so0, so1)
        wid = lax.axis_index("s") * nc + lax.axis_index("c")
        base = wid * rpw
        pltpu.sync_copy(t_h.at[pl.ds(base, rpw)], t_v)
        pltpu.sync_copy(sa_h, sa_v)
        pltpu.sync_copy(sb_h, sb_v)

        def start_in(c):
            b = c % 2
            rb = base + c * rc
            hz = pltpu.async_copy(z0_h.at[pl.ds(rb, rc)], z0v[b], sz[b])
            hn = pltpu.async_copy(nz_h.at[pl.ds(rb, rc)], nzv[b], sn[b])
            return (hz, hn)

        def start_out(c):
            b = c % 2
            rb = base + c * rc
            return pltpu.async_copy(ov[b], out_h.at[pl.ds(rb, rc)], so[b])

        def compute(c):
            b = c % 2
            z0b, nzb, ob = z0v[b], nzv[b], ov[b]

            @plsc.parallel_loop(0, rc, unroll=4)
            def _row(r):
                tb = plsc.load_gather(
                    t_v, [jnp.full((lanes,), c * rc + r, jnp.int32)])
                ab = plsc.load_gather(sa_v, [tb])
                bb = plsc.load_gather(sb_v, [tb])
                for j in range(nvec):
                    s = pl.ds(j * lanes, lanes)
                    ob[r, s] = ab * z0b[r, s] + bb * nzb[r, s]

        in_h = {}
        out_h_d = {}
        in_h[0] = start_in(0)
        if nchunks > 1:
            in_h[1] = start_in(1)
        for c in range(nchunks):
            if c >= 2:
                out_h_d[c - 2].wait()
            in_h[c][0].wait()
            in_h[c][1].wait()
            compute(c)
            out_h_d[c] = start_out(c)
            if c + 2 < nchunks:
                in_h[c + 2] = start_in(c + 2)
        for c in range(max(0, nchunks - 2), nchunks):
            out_h_d[c].wait()

    return run(z0, t_n, noise, sa, sb)


def kernel(z0_nk, t_n, noise, alpha_bar):
    sa, sb = _sqrt_tables(alpha_bar)
    z_t = _sc_scale_add(z0_nk, t_n.astype(jnp.int32), noise, sa, sb)
    return (z_t, noise)
